# baseline (speedup 1.0000x reference)
import jax
import jax.numpy as jnp
from jax import lax
from jax.experimental import pallas as pl
from jax.experimental.pallas import tpu as pltpu

SIZE = 1000000
DIM = 32
BATCH = 16384

REG = BATCH + 256
TILE_END = 999936
ANCHOR_CAP = TILE_END - REG
EDGE = SIZE - TILE_END


COPY_W = 51200
COPY_BLOCKS = (SIZE + COPY_W - 1) // COPY_W


def _copy_body(in_ref, out_ref):
  out_ref[...] = in_ref[...]


def _merge_region(out_ref, regbuf, sem, start, width, rolled, mask):
  rd = pltpu.make_async_copy(
      out_ref.at[:, pl.ds(start, width)], regbuf.at[:, pl.ds(0, width)], sem)
  rd.start()
  rd.wait()
  regbuf[:, pl.ds(0, width)] = jnp.where(
      mask, rolled, regbuf[:, pl.ds(0, width)])
  wr = pltpu.make_async_copy(
      regbuf.at[:, pl.ds(0, width)], out_ref.at[:, pl.ds(start, width)], sem)
  wr.start()
  wr.wait()


def _tc_body(ptr_ref, zT_ref, bank_in_ref, outT_ref, znbuf, regbuf, w_sem):
  del bank_in_ref
  zt = zT_ref[...]
  norm = jnp.sqrt(jnp.sum(zt * zt, axis=0, keepdims=True))
  znbuf[:, pl.ds(0, BATCH)] = zt / jnp.maximum(norm, 1e-12)

  s = jnp.remainder(ptr_ref[0], SIZE)
  lane = lax.broadcasted_iota(jnp.int32, (DIM, REG), 1)
  no_wrap = s + BATCH <= SIZE

  @pl.when(no_wrap)
  def _():
    a = jnp.minimum((s // 128) * 128, ANCHOR_CAP)
    a = pl.multiple_of(a, 128)
    r = s - a
    rolled = pltpu.roll(znbuf[...], r, axis=1)
    mask = jnp.logical_and(lane >= r, lane < r + BATCH)
    _merge_region(outT_ref, regbuf, w_sem, a, REG, rolled, mask)

  @pl.when(jnp.logical_not(no_wrap))
  def _():
    r_t = s - ANCHOR_CAP
    rolled_t = pltpu.roll(znbuf[...], jnp.remainder(r_t, REG), axis=1)
    mask_t = lane >= r_t
    _merge_region(outT_ref, regbuf, w_sem, ANCHOR_CAP, REG, rolled_t, mask_t)
    b1 = s + BATCH - SIZE
    rolled_h = pltpu.roll(znbuf[:, pl.ds(0, BATCH)], b1, axis=1)
    mask_h = lane[:, :BATCH] < b1
    _merge_region(outT_ref, regbuf, w_sem, 0, BATCH, rolled_h, mask_h)

def _edge_body(ptr_ref, zT_ref, in_ref, out_ref):
  s = jnp.remainder(ptr_ref[0], SIZE)
  se = s - TILE_END
  zt = zT_ref[...]
  norm = jnp.sqrt(jnp.sum(zt * zt, axis=0, keepdims=True))
  zn = zt / jnp.maximum(norm, 1e-12)
  rolled = pltpu.roll(zn, jnp.remainder(se, BATCH), axis=1)[:, :128]
  lane = lax.broadcasted_iota(jnp.int32, (DIM, 128), 1)
  mask = jnp.logical_and(lane >= se, lane < se + BATCH)
  out_ref[...] = jnp.where(mask, rolled, in_ref[...])


def kernel(z, bank, ptr):
  zT = z.T
  bankT = bank.T
  bank_copy = pl.pallas_call(
      _copy_body,
      grid=(COPY_BLOCKS,),
      in_specs=[pl.BlockSpec((DIM, COPY_W), lambda i: (0, i))],
      out_specs=pl.BlockSpec((DIM, COPY_W), lambda i: (0, i)),
      out_shape=jax.ShapeDtypeStruct((DIM, SIZE), jnp.float32),
      name="bank_copy",
  )(bankT)
  outT = pl.pallas_call(
      _tc_body,
      in_specs=[
          pl.BlockSpec(memory_space=pltpu.SMEM),
          pl.BlockSpec(memory_space=pltpu.VMEM),
          pl.BlockSpec(memory_space=pl.ANY),
      ],
      out_specs=pl.BlockSpec(memory_space=pl.ANY),
      out_shape=jax.ShapeDtypeStruct((DIM, SIZE), jnp.float32),
      input_output_aliases={2: 0},
      scratch_shapes=[
          pltpu.VMEM((DIM, REG), jnp.float32),
          pltpu.VMEM((DIM, REG), jnp.float32),
          pltpu.SemaphoreType.DMA,
      ],
      name="bank_window_write",
  )(ptr, zT, bank_copy)
  outT = pl.pallas_call(
      _edge_body,
      grid=(1,),
      in_specs=[
          pl.BlockSpec(memory_space=pltpu.SMEM),
          pl.BlockSpec((DIM, BATCH), lambda i: (0, 0)),
          pl.BlockSpec((DIM, 128), lambda i: (0, TILE_END // 128)),
      ],
      out_specs=pl.BlockSpec((DIM, 128), lambda i: (0, TILE_END // 128)),
      out_shape=jax.ShapeDtypeStruct((DIM, SIZE), jnp.float32),
      input_output_aliases={2: 0},
      name="bank_edge_fix",
  )(ptr, zT, outT)
  new_bank = outT.T
  p = ptr[0]
  new_ptr = (p + BATCH) % SIZE
  wrapped = jnp.logical_or(new_ptr < p, p + BATCH >= SIZE)
  return new_bank, jnp.array([new_ptr], dtype=jnp.int32), jnp.reshape(wrapped, (1,))


# speedup vs baseline: 1.0128x; 1.0128x over previous
import jax
import jax.numpy as jnp
from jax import lax
from jax.experimental import pallas as pl
from jax.experimental.pallas import tpu as pltpu

SIZE = 1000000
DIM = 32
BATCH = 16384

REG = BATCH + 256
TILE_END = 999936
ANCHOR_CAP = TILE_END - REG
EDGE = SIZE - TILE_END


COPY_W = 128000
COPY_BLOCKS = (SIZE + COPY_W - 1) // COPY_W


def _copy_body(in_ref, out_ref):
  out_ref[...] = in_ref[...]


def _merge_region(out_ref, regbuf, sem, start, width, rolled, mask):
  rd = pltpu.make_async_copy(
      out_ref.at[:, pl.ds(start, width)], regbuf.at[:, pl.ds(0, width)], sem)
  rd.start()
  rd.wait()
  regbuf[:, pl.ds(0, width)] = jnp.where(
      mask, rolled, regbuf[:, pl.ds(0, width)])
  wr = pltpu.make_async_copy(
      regbuf.at[:, pl.ds(0, width)], out_ref.at[:, pl.ds(start, width)], sem)
  wr.start()
  wr.wait()


def _tc_body(ptr_ref, zT_ref, bank_in_ref, outT_ref, znbuf, regbuf, w_sem):
  del bank_in_ref
  zt = zT_ref[...]
  norm = jnp.sqrt(jnp.sum(zt * zt, axis=0, keepdims=True))
  znbuf[:, pl.ds(0, BATCH)] = zt / jnp.maximum(norm, 1e-12)

  s = jnp.remainder(ptr_ref[0], SIZE)
  lane = lax.broadcasted_iota(jnp.int32, (DIM, REG), 1)
  no_wrap = s + BATCH <= SIZE

  @pl.when(no_wrap)
  def _():
    a = jnp.minimum((s // 128) * 128, ANCHOR_CAP)
    a = pl.multiple_of(a, 128)
    r = s - a
    rolled = pltpu.roll(znbuf[...], r, axis=1)
    mask = jnp.logical_and(lane >= r, lane < r + BATCH)
    _merge_region(outT_ref, regbuf, w_sem, a, REG, rolled, mask)

  @pl.when(jnp.logical_not(no_wrap))
  def _():
    r_t = s - ANCHOR_CAP
    rolled_t = pltpu.roll(znbuf[...], jnp.remainder(r_t, REG), axis=1)
    mask_t = lane >= r_t
    _merge_region(outT_ref, regbuf, w_sem, ANCHOR_CAP, REG, rolled_t, mask_t)
    b1 = s + BATCH - SIZE
    rolled_h = pltpu.roll(znbuf[:, pl.ds(0, BATCH)], b1, axis=1)
    mask_h = lane[:, :BATCH] < b1
    _merge_region(outT_ref, regbuf, w_sem, 0, BATCH, rolled_h, mask_h)

def _edge_body(ptr_ref, zT_ref, in_ref, out_ref):
  s = jnp.remainder(ptr_ref[0], SIZE)
  se = s - TILE_END
  zt = zT_ref[...]
  norm = jnp.sqrt(jnp.sum(zt * zt, axis=0, keepdims=True))
  zn = zt / jnp.maximum(norm, 1e-12)
  rolled = pltpu.roll(zn, jnp.remainder(se, BATCH), axis=1)[:, :128]
  lane = lax.broadcasted_iota(jnp.int32, (DIM, 128), 1)
  mask = jnp.logical_and(lane >= se, lane < se + BATCH)
  out_ref[...] = jnp.where(mask, rolled, in_ref[...])


def kernel(z, bank, ptr):
  zT = z.T
  bankT = bank.T
  bank_copy = pl.pallas_call(
      _copy_body,
      grid=(COPY_BLOCKS,),
      in_specs=[pl.BlockSpec((DIM, COPY_W), lambda i: (0, i))],
      out_specs=pl.BlockSpec((DIM, COPY_W), lambda i: (0, i)),
      out_shape=jax.ShapeDtypeStruct((DIM, SIZE), jnp.float32),
      compiler_params=pltpu.CompilerParams(vmem_limit_bytes=100 * 1024 * 1024),
      name="bank_copy",
  )(bankT)
  outT = pl.pallas_call(
      _tc_body,
      in_specs=[
          pl.BlockSpec(memory_space=pltpu.SMEM),
          pl.BlockSpec(memory_space=pltpu.VMEM),
          pl.BlockSpec(memory_space=pl.ANY),
      ],
      out_specs=pl.BlockSpec(memory_space=pl.ANY),
      out_shape=jax.ShapeDtypeStruct((DIM, SIZE), jnp.float32),
      input_output_aliases={2: 0},
      scratch_shapes=[
          pltpu.VMEM((DIM, REG), jnp.float32),
          pltpu.VMEM((DIM, REG), jnp.float32),
          pltpu.SemaphoreType.DMA,
      ],
      name="bank_window_write",
  )(ptr, zT, bank_copy)
  outT = pl.pallas_call(
      _edge_body,
      grid=(1,),
      in_specs=[
          pl.BlockSpec(memory_space=pltpu.SMEM),
          pl.BlockSpec((DIM, BATCH), lambda i: (0, 0)),
          pl.BlockSpec((DIM, 128), lambda i: (0, TILE_END // 128)),
      ],
      out_specs=pl.BlockSpec((DIM, 128), lambda i: (0, TILE_END // 128)),
      out_shape=jax.ShapeDtypeStruct((DIM, SIZE), jnp.float32),
      input_output_aliases={2: 0},
      name="bank_edge_fix",
  )(ptr, zT, outT)
  new_bank = outT.T
  p = ptr[0]
  new_ptr = (p + BATCH) % SIZE
  wrapped = jnp.logical_or(new_ptr < p, p + BATCH >= SIZE)
  return new_bank, jnp.array([new_ptr], dtype=jnp.int32), jnp.reshape(wrapped, (1,))


# speedup vs baseline: 1.0960x; 1.0821x over previous
import jax
import jax.numpy as jnp
from jax import lax
from jax.experimental import pallas as pl
from jax.experimental.pallas import tpu as pltpu

SIZE = 1000000
DIM = 32
BATCH = 16384

COPY_W = 102400
COPY_BLOCKS = (SIZE + COPY_W - 1) // COPY_W
SUB = BATCH + 128
SUB_CAP = COPY_W - SUB


def _body(ptr_ref, zT_ref, bank_ref, out_ref, znbuf):
  i = pl.program_id(0)

  @pl.when(i == 0)
  def _():
    zt = zT_ref[...]
    norm = jnp.sqrt(jnp.sum(zt * zt, axis=0, keepdims=True))
    znbuf[...] = zt / jnp.maximum(norm, 1e-12)

  out_ref[...] = bank_ref[...]

  s = jnp.remainder(ptr_ref[0], SIZE)
  g0 = i * COPY_W
  d0 = jnp.remainder(g0 - s, SIZE)

  @pl.when(d0 < BATCH)
  def _():
    rolled = pltpu.roll(znbuf[...], jnp.remainder(-d0, BATCH), axis=1)
    lane = lax.broadcasted_iota(jnp.int32, (DIM, BATCH), 1)
    mask = lane < BATCH - d0
    out_ref[:, :BATCH] = jnp.where(mask, rolled, out_ref[:, :BATCH])

  sg = s - g0

  @pl.when(jnp.logical_and(sg > 0, sg < COPY_W))
  def _():
    u0 = jnp.minimum((sg // 128) * 128, SUB_CAP)
    u0 = pl.multiple_of(u0, 128)
    r = sg - u0
    ext = jnp.concatenate(
        [znbuf[...], jnp.zeros((DIM, SUB - BATCH), jnp.float32)], axis=1)
    rolled = pltpu.roll(ext, r, axis=1)
    lane = lax.broadcasted_iota(jnp.int32, (DIM, SUB), 1)
    mask = jnp.logical_and(lane >= r, lane < r + BATCH)
    cur = out_ref[:, pl.ds(u0, SUB)]
    out_ref[:, pl.ds(u0, SUB)] = jnp.where(mask, rolled, cur)


def kernel(z, bank, ptr):
  zT = z.T
  bankT = bank.T
  outT = pl.pallas_call(
      _body,
      grid=(COPY_BLOCKS,),
      in_specs=[
          pl.BlockSpec(memory_space=pltpu.SMEM),
          pl.BlockSpec((DIM, BATCH), lambda i: (0, 0)),
          pl.BlockSpec((DIM, COPY_W), lambda i: (0, i)),
      ],
      out_specs=pl.BlockSpec((DIM, COPY_W), lambda i: (0, i)),
      out_shape=jax.ShapeDtypeStruct((DIM, SIZE), jnp.float32),
      scratch_shapes=[pltpu.VMEM((DIM, BATCH), jnp.float32)],
      compiler_params=pltpu.CompilerParams(
          vmem_limit_bytes=63 * 1024 * 1024),
      name="bank_enqueue",
  )(ptr, zT, bankT)
  new_bank = outT.T
  p = ptr[0]
  new_ptr = (p + BATCH) % SIZE
  wrapped = jnp.logical_or(new_ptr < p, p + BATCH >= SIZE)
  return new_bank, jnp.array([new_ptr], dtype=jnp.int32), jnp.reshape(wrapped, (1,))
